# baseline (speedup 1.0000x reference)
import functools
import math

import jax
import jax.numpy as jnp
from jax import lax
from jax.experimental import pallas as pl
from jax.experimental.pallas import tpu as pltpu
from jax.experimental.pallas import tpu_sc as plsc

N_ENT = 4096
N_REL = 512
N_TRI = 131072
DIM = 128
LANES = 16
NC = 2
NS = 16
NW = NC * NS
TRI_PER_W = N_TRI // NW
CHUNK = 128
N_CHUNK = TRI_PER_W // CHUNK
DENOM_INV = 1.0 / (3.0 * math.sqrt(DIM))

ROWS_PER_SC = N_ENT // NC
PASS_ROWS = 256
N_PASS = ROWS_PER_SC // PASS_ROWS
ACC = PASS_ROWS * N_ENT
TRI_PER_T = N_TRI // NS
STRIPE = ACC // NS
ZCHUNK = 16384

_mesh = plsc.VectorSubcoreMesh(core_axis_name="c", subcore_axis_name="s")


def _newton_sqrt(x):
    ib = lax.bitcast_convert_type(x, jnp.int32)
    ib = jnp.int32(0x5F3759DF) - lax.shift_right_arithmetic(ib, 1)
    y = lax.bitcast_convert_type(ib, jnp.float32)
    for _ in range(3):
        y = y * (1.5 - 0.5 * x * y * y)
    return x * y


def _score_body(ent_hbm, rel_hbm, h_hbm, t_hbm, r_hbm, scores_hbm,
                hidx, tidx, ridx, hrows0, trows0, rrows0,
                hrows1, trows1, rrows1, scorebuf, sem0, sem1):
    cid = lax.axis_index("c")
    sid = lax.axis_index("s")
    wid = sid * NC + cid
    rowbase = wid * (TRI_PER_W // 128)

    lane = lax.broadcasted_iota(jnp.int32, (LANES,), 0)
    perms = [jnp.mod(lane + s, LANES).reshape(LANES, 1) for s in (8, 4, 2, 1)]
    dnums = lax.GatherDimensionNumbers(
        offset_dims=(), collapsed_slice_dims=(0,), start_index_map=(0,))

    def _permute(x, perm):
        return lax.gather(x, perm, dnums, slice_sizes=(1,),
                          mode=lax.GatherScatterMode.PROMISE_IN_BOUNDS)

    for side in range(2):
        ent_s = ent_hbm.at[side]
        rel_s = rel_hbm.at[side]
        pltpu.sync_copy(h_hbm.at[side].at[pl.ds(rowbase, N_CHUNK)], hidx)
        pltpu.sync_copy(t_hbm.at[side].at[pl.ds(rowbase, N_CHUNK)], tidx)
        pltpu.sync_copy(r_hbm.at[side].at[pl.ds(rowbase, N_CHUNK)], ridx)

        def _fire(cidx, hrows, trows, rrows, sem):
            pltpu.async_copy(ent_s.at[hidx.at[cidx]], hrows, sem)
            pltpu.async_copy(ent_s.at[tidx.at[cidx]], trows, sem)
            pltpu.async_copy(rel_s.at[ridx.at[cidx]], rrows, sem)

        def _drain(cidx, hrows, trows, rrows, sem):
            pltpu.make_async_copy(ent_s.at[hidx.at[cidx]], hrows, sem).wait()
            pltpu.make_async_copy(ent_s.at[tidx.at[cidx]], trows, sem).wait()
            pltpu.make_async_copy(rel_s.at[ridx.at[cidx]], rrows, sem).wait()

        def _compute(cidx, hrows, trows, rrows):
            @plsc.parallel_loop(0, CHUNK // LANES)
            def _group(g):
                vec = jnp.zeros((LANES,), jnp.float32)
                for u in range(LANES):
                    i = g * LANES + u
                    acc = jnp.zeros((LANES,), jnp.float32)
                    for j in range(DIM // LANES):
                        dh = hrows[i, pl.ds(j * LANES, LANES)]
                        dr = rrows[i, pl.ds(j * LANES, LANES)]
                        dt = trows[i, pl.ds(j * LANES, LANES)]
                        d = dh + dr - dt
                        acc = acc + d * d
                    for perm in perms:
                        acc = acc + _permute(acc, perm)
                    vec = jnp.where(lane == u, acc, vec)
                score = 1.0 - _newton_sqrt(vec) * DENOM_INV
                scorebuf[pl.ds(cidx * CHUNK + g * LANES, LANES)] = score

        _fire(0, hrows0, trows0, rrows0, sem0)

        @pl.loop(0, N_CHUNK // 2)
        def _chunk(k):
            c0 = 2 * k
            _fire(c0 + 1, hrows1, trows1, rrows1, sem1)
            _drain(c0, hrows0, trows0, rrows0, sem0)
            _compute(c0, hrows0, trows0, rrows0)

            @pl.when(k < N_CHUNK // 2 - 1)
            def _():
                _fire(c0 + 2, hrows0, trows0, rrows0, sem0)

            _drain(c0 + 1, hrows1, trows1, rrows1, sem1)
            _compute(c0 + 1, hrows1, trows1, rrows1)

        pltpu.sync_copy(scorebuf,
                        scores_hbm.at[side].at[pl.ds(wid * TRI_PER_W,
                                                     TRI_PER_W)])


_score_call = pl.kernel(
    _score_body,
    out_type=jax.ShapeDtypeStruct((2, N_TRI), jnp.float32),
    mesh=_mesh,
    scratch_types=[
        pltpu.VMEM((N_CHUNK, CHUNK), jnp.int32),
        pltpu.VMEM((N_CHUNK, CHUNK), jnp.int32),
        pltpu.VMEM((N_CHUNK, CHUNK), jnp.int32),
        pltpu.VMEM((CHUNK, DIM), jnp.float32),
        pltpu.VMEM((CHUNK, DIM), jnp.float32),
        pltpu.VMEM((CHUNK, DIM), jnp.float32),
        pltpu.VMEM((CHUNK, DIM), jnp.float32),
        pltpu.VMEM((CHUNK, DIM), jnp.float32),
        pltpu.VMEM((CHUNK, DIM), jnp.float32),
        pltpu.VMEM((TRI_PER_W,), jnp.float32),
        pltpu.SemaphoreType.DMA,
        pltpu.SemaphoreType.DMA,
    ],
)


N_SCHUNK = TRI_PER_T // CHUNK


def _scatter_body(h_hbm, t_hbm, s_hbm, out_hbm,
                  acc, hbuf, gidx, vals, idxbuf, zeros_v, sem, zsem):
    cid = lax.axis_index("c")
    sid = lax.axis_index("s")
    tb = sid * (TRI_PER_T // CHUNK)

    lane = lax.broadcasted_iota(jnp.int32, (LANES,), 0)
    dump = jnp.int32(ACC) + lane * 8

    @pl.loop(0, ZCHUNK // LANES)
    def _zinit(k):
        zeros_v[pl.ds(k * LANES, LANES)] = jnp.zeros((LANES,), jnp.float32)

    for side in range(2):
        cp0 = pltpu.async_copy(h_hbm.at[side].at[pl.ds(tb, N_SCHUNK)],
                               hbuf, sem)
        cp1 = pltpu.async_copy(t_hbm.at[side].at[pl.ds(tb, N_SCHUNK)],
                               gidx, zsem)
        cp2 = pltpu.async_copy(s_hbm.at[side].at[pl.ds(tb, N_SCHUNK)],
                               vals, sem)
        cp0.wait()
        cp1.wait()
        cp2.wait()

        @plsc.parallel_loop(0, N_SCHUNK)
        def _pre(j):
            for g in range(CHUNK // LANES):
                hv = hbuf[j, pl.ds(g * LANES, LANES)]
                tv = gidx[j, pl.ds(g * LANES, LANES)]
                gidx[j, pl.ds(g * LANES, LANES)] = hv * N_ENT + tv

        @pl.loop(0, N_PASS)
        def _pass(p):
            base = cid * (ROWS_PER_SC * N_ENT) + p * (PASS_ROWS * N_ENT)

            zcps = [pltpu.async_copy(
                        zeros_v,
                        acc.at[pl.ds(sid * STRIPE + z * ZCHUNK, ZCHUNK)],
                        zsem)
                    for z in range(STRIPE // ZCHUNK)]

            @plsc.parallel_loop(0, N_SCHUNK)
            def _idx(j):
                for g in range(CHUNK // LANES):
                    gv = gidx[j, pl.ds(g * LANES, LANES)] - base
                    m = (gv >= 0) & (gv < ACC)
                    idxbuf[j, pl.ds(g * LANES, LANES)] = jnp.where(m, gv,
                                                                   dump)

            for cp in zcps:
                cp.wait()
            plsc.subcore_barrier()

            cps = [pltpu.async_copy(vals.at[j], acc.at[idxbuf.at[j]], sem,
                                    add=True)
                   for j in range(N_SCHUNK)]
            for cp in cps:
                cp.wait()

            plsc.subcore_barrier()
            pl.delay(2000)
            pltpu.sync_copy(acc.at[pl.ds(sid * STRIPE, STRIPE)],
                            out_hbm.at[side].at[pl.ds(base + sid * STRIPE,
                                                      STRIPE)])


_scatter_call = pl.kernel(
    _scatter_body,
    out_type=jax.ShapeDtypeStruct((2, N_ENT * N_ENT), jnp.float32),
    mesh=_mesh,
    scratch_types=[
        pltpu.VMEM_SHARED((ACC + 128,), jnp.float32),
        pltpu.VMEM((N_SCHUNK, CHUNK), jnp.int32),
        pltpu.VMEM((N_SCHUNK, CHUNK), jnp.int32),
        pltpu.VMEM((N_SCHUNK, CHUNK), jnp.float32),
        pltpu.VMEM((N_SCHUNK, CHUNK), jnp.int32),
        pltpu.VMEM((ZCHUNK,), jnp.float32),
        pltpu.SemaphoreType.DMA,
        pltpu.SemaphoreType.DMA,
    ],
)


def kernel(entity_emb_sr, entity_emb_tg, relation_emb_sr, relation_emb_tg,
           head_sr, tail_sr, relation_sr, head_tg, tail_tg, relation_tg):
    ent = jnp.stack([entity_emb_sr, entity_emb_tg])
    rel = jnp.stack([relation_emb_sr, relation_emb_tg])

    def _idx2(a, b):
        return jnp.stack([a.astype(jnp.int32).reshape(N_TRI // 128, 128),
                          b.astype(jnp.int32).reshape(N_TRI // 128, 128)])

    h2 = _idx2(head_sr, head_tg)
    t2 = _idx2(tail_sr, tail_tg)
    r2 = _idx2(relation_sr, relation_tg)
    scores = _score_call(ent, rel, h2, t2, r2)
    out = _scatter_call(h2, t2, scores.reshape(2, N_TRI // 128, 128))
    return (out[0].reshape(N_ENT, N_ENT), out[1].reshape(N_ENT, N_ENT))


# speedup vs baseline: 1.4800x; 1.4800x over previous
import functools
import math

import jax
import jax.numpy as jnp
from jax import lax
from jax.experimental import pallas as pl
from jax.experimental.pallas import tpu as pltpu
from jax.experimental.pallas import tpu_sc as plsc

N_ENT = 4096
N_REL = 512
N_TRI = 131072
DIM = 128
LANES = 16
NC = 2
NS = 16
NW = NC * NS
TRI_PER_W = N_TRI // NW
CHUNK = 128
N_CHUNK = TRI_PER_W // CHUNK
DENOM_INV = 1.0 / (3.0 * math.sqrt(DIM))

ROWS_PER_SC = N_ENT // NC
PASS_ROWS = 256
N_PASS = ROWS_PER_SC // PASS_ROWS
ACC = PASS_ROWS * N_ENT
TRI_PER_T = N_TRI // NS
STRIPE = ACC // NS
ZCHUNK = 16384

_mesh = plsc.VectorSubcoreMesh(core_axis_name="c", subcore_axis_name="s")


def _newton_sqrt(x):
    ib = lax.bitcast_convert_type(x, jnp.int32)
    ib = jnp.int32(0x5F3759DF) - lax.shift_right_arithmetic(ib, 1)
    y = lax.bitcast_convert_type(ib, jnp.float32)
    for _ in range(3):
        y = y * (1.5 - 0.5 * x * y * y)
    return x * y


def _score_body(ent_a, ent_b, rel_a, rel_b, h_a, h_b, t_a, t_b, r_a, r_b,
                scores_a, scores_b,
                hidx, tidx, ridx, hrows0, trows0, rrows0,
                hrows1, trows1, rrows1, scorebuf, sem0, sem1):
    cid = lax.axis_index("c")
    sid = lax.axis_index("s")
    wid = sid * NC + cid
    rowbase = wid * (TRI_PER_W // 128)

    lane = lax.broadcasted_iota(jnp.int32, (LANES,), 0)
    perms = [jnp.mod(lane + s, LANES).reshape(LANES, 1) for s in (8, 4, 2, 1)]
    dnums = lax.GatherDimensionNumbers(
        offset_dims=(), collapsed_slice_dims=(0,), start_index_map=(0,))

    def _permute(x, perm):
        return lax.gather(x, perm, dnums, slice_sizes=(1,),
                          mode=lax.GatherScatterMode.PROMISE_IN_BOUNDS)

    for ent_s, rel_s, h_s, t_s, r_s, scores_s in (
            (ent_a, rel_a, h_a, t_a, r_a, scores_a),
            (ent_b, rel_b, h_b, t_b, r_b, scores_b)):
        pltpu.sync_copy(h_s.at[pl.ds(rowbase, N_CHUNK)], hidx)
        pltpu.sync_copy(t_s.at[pl.ds(rowbase, N_CHUNK)], tidx)
        pltpu.sync_copy(r_s.at[pl.ds(rowbase, N_CHUNK)], ridx)

        def _fire(cidx, hrows, trows, rrows, sem):
            pltpu.async_copy(ent_s.at[hidx.at[cidx]], hrows, sem)
            pltpu.async_copy(ent_s.at[tidx.at[cidx]], trows, sem)
            pltpu.async_copy(rel_s.at[ridx.at[cidx]], rrows, sem)

        def _drain(cidx, hrows, trows, rrows, sem):
            pltpu.make_async_copy(ent_s.at[hidx.at[cidx]], hrows, sem).wait()
            pltpu.make_async_copy(ent_s.at[tidx.at[cidx]], trows, sem).wait()
            pltpu.make_async_copy(rel_s.at[ridx.at[cidx]], rrows, sem).wait()

        def _compute(cidx, hrows, trows, rrows):
            @plsc.parallel_loop(0, CHUNK // LANES)
            def _group(g):
                vec = jnp.zeros((LANES,), jnp.float32)
                for u in range(LANES):
                    i = g * LANES + u
                    acc = jnp.zeros((LANES,), jnp.float32)
                    for j in range(DIM // LANES):
                        dh = hrows[i, pl.ds(j * LANES, LANES)]
                        dr = rrows[i, pl.ds(j * LANES, LANES)]
                        dt = trows[i, pl.ds(j * LANES, LANES)]
                        d = dh + dr - dt
                        acc = acc + d * d
                    for perm in perms:
                        acc = acc + _permute(acc, perm)
                    vec = jnp.where(lane == u, acc, vec)
                score = 1.0 - _newton_sqrt(vec) * DENOM_INV
                scorebuf[pl.ds(cidx * CHUNK + g * LANES, LANES)] = score

        _fire(0, hrows0, trows0, rrows0, sem0)

        @pl.loop(0, N_CHUNK // 2)
        def _chunk(k):
            c0 = 2 * k
            _fire(c0 + 1, hrows1, trows1, rrows1, sem1)
            _drain(c0, hrows0, trows0, rrows0, sem0)
            _compute(c0, hrows0, trows0, rrows0)

            @pl.when(k < N_CHUNK // 2 - 1)
            def _():
                _fire(c0 + 2, hrows0, trows0, rrows0, sem0)

            _drain(c0 + 1, hrows1, trows1, rrows1, sem1)
            _compute(c0 + 1, hrows1, trows1, rrows1)

        pltpu.sync_copy(scorebuf,
                        scores_s.at[pl.ds(wid * TRI_PER_W, TRI_PER_W)])


_score_call = pl.kernel(
    _score_body,
    out_type=(jax.ShapeDtypeStruct((N_TRI,), jnp.float32),
              jax.ShapeDtypeStruct((N_TRI,), jnp.float32)),
    mesh=_mesh,
    scratch_types=[
        pltpu.VMEM((N_CHUNK, CHUNK), jnp.int32),
        pltpu.VMEM((N_CHUNK, CHUNK), jnp.int32),
        pltpu.VMEM((N_CHUNK, CHUNK), jnp.int32),
        pltpu.VMEM((CHUNK, DIM), jnp.float32),
        pltpu.VMEM((CHUNK, DIM), jnp.float32),
        pltpu.VMEM((CHUNK, DIM), jnp.float32),
        pltpu.VMEM((CHUNK, DIM), jnp.float32),
        pltpu.VMEM((CHUNK, DIM), jnp.float32),
        pltpu.VMEM((CHUNK, DIM), jnp.float32),
        pltpu.VMEM((TRI_PER_W,), jnp.float32),
        pltpu.SemaphoreType.DMA,
        pltpu.SemaphoreType.DMA,
    ],
)


N_SCHUNK = TRI_PER_T // CHUNK


def _scatter_body(h_a, h_b, t_a, t_b, s_a, s_b, out_a, out_b,
                  acc, hbuf, gidx, vals, idxbuf, zeros_v, sem, zsem):
    cid = lax.axis_index("c")
    sid = lax.axis_index("s")
    tb = sid * (TRI_PER_T // CHUNK)

    lane = lax.broadcasted_iota(jnp.int32, (LANES,), 0)
    dump = jnp.int32(ACC) + lane * 8

    @pl.loop(0, ZCHUNK // LANES)
    def _zinit(k):
        zeros_v[pl.ds(k * LANES, LANES)] = jnp.zeros((LANES,), jnp.float32)

    for h_s, t_s, s_s, out_s in ((h_a, t_a, s_a, out_a),
                                 (h_b, t_b, s_b, out_b)):
        cp0 = pltpu.async_copy(h_s.at[pl.ds(tb, N_SCHUNK)], hbuf, sem)
        cp1 = pltpu.async_copy(t_s.at[pl.ds(tb, N_SCHUNK)], gidx, zsem)
        cp2 = pltpu.async_copy(s_s.at[pl.ds(tb, N_SCHUNK)], vals, sem)
        cp0.wait()
        cp1.wait()
        cp2.wait()

        @plsc.parallel_loop(0, N_SCHUNK)
        def _pre(j):
            for g in range(CHUNK // LANES):
                hv = hbuf[j, pl.ds(g * LANES, LANES)]
                tv = gidx[j, pl.ds(g * LANES, LANES)]
                gidx[j, pl.ds(g * LANES, LANES)] = hv * N_ENT + tv

        @pl.loop(0, N_PASS)
        def _pass(p):
            base = cid * (ROWS_PER_SC * N_ENT) + p * (PASS_ROWS * N_ENT)

            zcps = [pltpu.async_copy(
                        zeros_v,
                        acc.at[pl.ds(sid * STRIPE + z * ZCHUNK, ZCHUNK)],
                        zsem)
                    for z in range(STRIPE // ZCHUNK)]

            @plsc.parallel_loop(0, N_SCHUNK)
            def _idx(j):
                for g in range(CHUNK // LANES):
                    gv = gidx[j, pl.ds(g * LANES, LANES)] - base
                    m = (gv >= 0) & (gv < ACC)
                    idxbuf[j, pl.ds(g * LANES, LANES)] = jnp.where(m, gv,
                                                                   dump)

            for cp in zcps:
                cp.wait()
            plsc.subcore_barrier()

            cps = [pltpu.async_copy(vals.at[j], acc.at[idxbuf.at[j]], sem,
                                    add=True)
                   for j in range(N_SCHUNK)]
            for cp in cps:
                cp.wait()

            plsc.subcore_barrier()
            pl.delay(2000)
            pltpu.sync_copy(acc.at[pl.ds(sid * STRIPE, STRIPE)],
                            out_s.at[pl.ds(base + sid * STRIPE, STRIPE)])


_scatter_call = pl.kernel(
    _scatter_body,
    out_type=(jax.ShapeDtypeStruct((N_ENT * N_ENT,), jnp.float32),
              jax.ShapeDtypeStruct((N_ENT * N_ENT,), jnp.float32)),
    mesh=_mesh,
    scratch_types=[
        pltpu.VMEM_SHARED((ACC + 128,), jnp.float32),
        pltpu.VMEM((N_SCHUNK, CHUNK), jnp.int32),
        pltpu.VMEM((N_SCHUNK, CHUNK), jnp.int32),
        pltpu.VMEM((N_SCHUNK, CHUNK), jnp.float32),
        pltpu.VMEM((N_SCHUNK, CHUNK), jnp.int32),
        pltpu.VMEM((ZCHUNK,), jnp.float32),
        pltpu.SemaphoreType.DMA,
        pltpu.SemaphoreType.DMA,
    ],
)


def kernel(entity_emb_sr, entity_emb_tg, relation_emb_sr, relation_emb_tg,
           head_sr, tail_sr, relation_sr, head_tg, tail_tg, relation_tg):
    def _i2(a):
        return a.astype(jnp.int32).reshape(N_TRI // 128, 128)

    ha, hb = _i2(head_sr), _i2(head_tg)
    ta, tb = _i2(tail_sr), _i2(tail_tg)
    ra, rb = _i2(relation_sr), _i2(relation_tg)
    sc_a, sc_b = _score_call(entity_emb_sr, entity_emb_tg,
                             relation_emb_sr, relation_emb_tg,
                             ha, hb, ta, tb, ra, rb)
    out_a, out_b = _scatter_call(ha, hb, ta, tb,
                                 sc_a.reshape(N_TRI // 128, 128),
                                 sc_b.reshape(N_TRI // 128, 128))
    return (out_a.reshape(N_ENT, N_ENT), out_b.reshape(N_ENT, N_ENT))


# speedup vs baseline: 1.7077x; 1.1539x over previous
import functools
import math

import jax
import jax.numpy as jnp
from jax import lax
from jax.experimental import pallas as pl
from jax.experimental.pallas import tpu as pltpu
from jax.experimental.pallas import tpu_sc as plsc

N_ENT = 4096
N_REL = 512
N_TRI = 131072
DIM = 128
LANES = 16
NC = 2
NS = 16
NW = NC * NS
TRI_PER_W = N_TRI // NW
CHUNK = 128
N_CHUNK = TRI_PER_W // CHUNK
DENOM_INV = 1.0 / (3.0 * math.sqrt(DIM))

ROWS_PER_SC = N_ENT // NC
PASS_ROWS = 256
N_PASS = ROWS_PER_SC // PASS_ROWS
ACC = PASS_ROWS * N_ENT
TRI_PER_T = N_TRI // NS
STRIPE = ACC // NS
ZCHUNK = 16384

_mesh = plsc.VectorSubcoreMesh(core_axis_name="c", subcore_axis_name="s")


def _newton_sqrt(x):
    ib = lax.bitcast_convert_type(x, jnp.int32)
    ib = jnp.int32(0x5F3759DF) - lax.shift_right_arithmetic(ib, 1)
    y = lax.bitcast_convert_type(ib, jnp.float32)
    for _ in range(3):
        y = y * (1.5 - 0.5 * x * y * y)
    return x * y


def _score_body(ent_a, ent_b, rel_a, rel_b, h_a, h_b, t_a, t_b, r_a, r_b,
                scores_a, scores_b,
                hidx, tidx, ridx, hrows0, trows0, rrows0,
                hrows1, trows1, rrows1, scorebuf, sem0, sem1):
    cid = lax.axis_index("c")
    sid = lax.axis_index("s")
    wid = sid * NC + cid
    rowbase = wid * (TRI_PER_W // 128)

    lane = lax.broadcasted_iota(jnp.int32, (LANES,), 0)
    perms = [jnp.mod(lane + s, LANES).reshape(LANES, 1) for s in (8, 4, 2, 1)]
    dnums = lax.GatherDimensionNumbers(
        offset_dims=(), collapsed_slice_dims=(0,), start_index_map=(0,))

    def _permute(x, perm):
        return lax.gather(x, perm, dnums, slice_sizes=(1,),
                          mode=lax.GatherScatterMode.PROMISE_IN_BOUNDS)

    for ent_s, rel_s, h_s, t_s, r_s, scores_s in (
            (ent_a, rel_a, h_a, t_a, r_a, scores_a),
            (ent_b, rel_b, h_b, t_b, r_b, scores_b)):
        pltpu.sync_copy(h_s.at[pl.ds(rowbase, N_CHUNK)], hidx)
        pltpu.sync_copy(t_s.at[pl.ds(rowbase, N_CHUNK)], tidx)
        pltpu.sync_copy(r_s.at[pl.ds(rowbase, N_CHUNK)], ridx)

        def _fire(cidx, hrows, trows, rrows, sem):
            pltpu.async_copy(ent_s.at[hidx.at[cidx]], hrows, sem)
            pltpu.async_copy(ent_s.at[tidx.at[cidx]], trows, sem)
            pltpu.async_copy(rel_s.at[ridx.at[cidx]], rrows, sem)

        def _drain(cidx, hrows, trows, rrows, sem):
            pltpu.make_async_copy(ent_s.at[hidx.at[cidx]], hrows, sem).wait()
            pltpu.make_async_copy(ent_s.at[tidx.at[cidx]], trows, sem).wait()
            pltpu.make_async_copy(rel_s.at[ridx.at[cidx]], rrows, sem).wait()

        def _compute(cidx, hrows, trows, rrows):
            @plsc.parallel_loop(0, CHUNK // LANES)
            def _group(g):
                @plsc.parallel_loop(0, LANES,
                                    carry=jnp.zeros((LANES,), jnp.float32))
                def vec(u, vec_c):
                    i = g * LANES + u
                    acc0 = jnp.zeros((LANES,), jnp.float32)
                    acc1 = jnp.zeros((LANES,), jnp.float32)
                    for j in range(0, DIM // LANES, 2):
                        dh0 = hrows[i, pl.ds(j * LANES, LANES)]
                        dr0 = rrows[i, pl.ds(j * LANES, LANES)]
                        dt0 = trows[i, pl.ds(j * LANES, LANES)]
                        dh1 = hrows[i, pl.ds((j + 1) * LANES, LANES)]
                        dr1 = rrows[i, pl.ds((j + 1) * LANES, LANES)]
                        dt1 = trows[i, pl.ds((j + 1) * LANES, LANES)]
                        d0 = (dh0 + dr0) - dt0
                        d1 = (dh1 + dr1) - dt1
                        acc0 = acc0 + d0 * d0
                        acc1 = acc1 + d1 * d1
                    acc = acc0 + acc1
                    for perm in perms:
                        acc = acc + _permute(acc, perm)
                    return jnp.where(lane == u, acc, vec_c)
                score = 1.0 - _newton_sqrt(vec) * DENOM_INV
                scorebuf[pl.ds(cidx * CHUNK + g * LANES, LANES)] = score

        _fire(0, hrows0, trows0, rrows0, sem0)

        @pl.loop(0, N_CHUNK // 2)
        def _chunk(k):
            c0 = 2 * k
            _fire(c0 + 1, hrows1, trows1, rrows1, sem1)
            _drain(c0, hrows0, trows0, rrows0, sem0)
            _compute(c0, hrows0, trows0, rrows0)

            @pl.when(k < N_CHUNK // 2 - 1)
            def _():
                _fire(c0 + 2, hrows0, trows0, rrows0, sem0)

            _drain(c0 + 1, hrows1, trows1, rrows1, sem1)
            _compute(c0 + 1, hrows1, trows1, rrows1)

        pltpu.sync_copy(scorebuf,
                        scores_s.at[pl.ds(wid * TRI_PER_W, TRI_PER_W)])


_score_call = pl.kernel(
    _score_body,
    out_type=(jax.ShapeDtypeStruct((N_TRI,), jnp.float32),
              jax.ShapeDtypeStruct((N_TRI,), jnp.float32)),
    mesh=_mesh,
    scratch_types=[
        pltpu.VMEM((N_CHUNK, CHUNK), jnp.int32),
        pltpu.VMEM((N_CHUNK, CHUNK), jnp.int32),
        pltpu.VMEM((N_CHUNK, CHUNK), jnp.int32),
        pltpu.VMEM((CHUNK, DIM), jnp.float32),
        pltpu.VMEM((CHUNK, DIM), jnp.float32),
        pltpu.VMEM((CHUNK, DIM), jnp.float32),
        pltpu.VMEM((CHUNK, DIM), jnp.float32),
        pltpu.VMEM((CHUNK, DIM), jnp.float32),
        pltpu.VMEM((CHUNK, DIM), jnp.float32),
        pltpu.VMEM((TRI_PER_W,), jnp.float32),
        pltpu.SemaphoreType.DMA,
        pltpu.SemaphoreType.DMA,
    ],
)


N_SCHUNK = TRI_PER_T // CHUNK


def _scatter_body(h_a, h_b, t_a, t_b, s_a, s_b, out_a, out_b,
                  acc, hbuf, gidx, vals, idxbuf, zeros_v, sem, zsem):
    cid = lax.axis_index("c")
    sid = lax.axis_index("s")
    tb = sid * (TRI_PER_T // CHUNK)

    lane = lax.broadcasted_iota(jnp.int32, (LANES,), 0)
    dump = jnp.int32(ACC) + lane * 8

    @pl.loop(0, ZCHUNK // LANES)
    def _zinit(k):
        zeros_v[pl.ds(k * LANES, LANES)] = jnp.zeros((LANES,), jnp.float32)

    for h_s, t_s, s_s, out_s in ((h_a, t_a, s_a, out_a),
                                 (h_b, t_b, s_b, out_b)):
        cp0 = pltpu.async_copy(h_s.at[pl.ds(tb, N_SCHUNK)], hbuf, sem)
        cp1 = pltpu.async_copy(t_s.at[pl.ds(tb, N_SCHUNK)], gidx, zsem)
        cp2 = pltpu.async_copy(s_s.at[pl.ds(tb, N_SCHUNK)], vals, sem)
        cp0.wait()
        cp1.wait()
        cp2.wait()

        @plsc.parallel_loop(0, N_SCHUNK)
        def _pre(j):
            for g in range(CHUNK // LANES):
                hv = hbuf[j, pl.ds(g * LANES, LANES)]
                tv = gidx[j, pl.ds(g * LANES, LANES)]
                gidx[j, pl.ds(g * LANES, LANES)] = hv * N_ENT + tv

        @pl.loop(0, N_PASS)
        def _pass(p):
            base = cid * (ROWS_PER_SC * N_ENT) + p * (PASS_ROWS * N_ENT)

            zcps = [pltpu.async_copy(
                        zeros_v,
                        acc.at[pl.ds(sid * STRIPE + z * ZCHUNK, ZCHUNK)],
                        zsem)
                    for z in range(STRIPE // ZCHUNK)]

            @plsc.parallel_loop(0, N_SCHUNK)
            def _idx(j):
                for g in range(CHUNK // LANES):
                    gv = gidx[j, pl.ds(g * LANES, LANES)] - base
                    m = (gv >= 0) & (gv < ACC)
                    idxbuf[j, pl.ds(g * LANES, LANES)] = jnp.where(m, gv,
                                                                   dump)

            for cp in zcps:
                cp.wait()
            plsc.subcore_barrier()

            cps = [pltpu.async_copy(vals.at[j], acc.at[idxbuf.at[j]], sem,
                                    add=True)
                   for j in range(N_SCHUNK)]
            for cp in cps:
                cp.wait()

            plsc.subcore_barrier()
            pl.delay(2000)
            pltpu.sync_copy(acc.at[pl.ds(sid * STRIPE, STRIPE)],
                            out_s.at[pl.ds(base + sid * STRIPE, STRIPE)])


_scatter_call = pl.kernel(
    _scatter_body,
    out_type=(jax.ShapeDtypeStruct((N_ENT * N_ENT,), jnp.float32),
              jax.ShapeDtypeStruct((N_ENT * N_ENT,), jnp.float32)),
    mesh=_mesh,
    scratch_types=[
        pltpu.VMEM_SHARED((ACC + 128,), jnp.float32),
        pltpu.VMEM((N_SCHUNK, CHUNK), jnp.int32),
        pltpu.VMEM((N_SCHUNK, CHUNK), jnp.int32),
        pltpu.VMEM((N_SCHUNK, CHUNK), jnp.float32),
        pltpu.VMEM((N_SCHUNK, CHUNK), jnp.int32),
        pltpu.VMEM((ZCHUNK,), jnp.float32),
        pltpu.SemaphoreType.DMA,
        pltpu.SemaphoreType.DMA,
    ],
)


def kernel(entity_emb_sr, entity_emb_tg, relation_emb_sr, relation_emb_tg,
           head_sr, tail_sr, relation_sr, head_tg, tail_tg, relation_tg):
    def _i2(a):
        return a.astype(jnp.int32).reshape(N_TRI // 128, 128)

    ha, hb = _i2(head_sr), _i2(head_tg)
    ta, tb = _i2(tail_sr), _i2(tail_tg)
    ra, rb = _i2(relation_sr), _i2(relation_tg)
    sc_a, sc_b = _score_call(entity_emb_sr, entity_emb_tg,
                             relation_emb_sr, relation_emb_tg,
                             ha, hb, ta, tb, ra, rb)
    out_a, out_b = _scatter_call(ha, hb, ta, tb,
                                 sc_a.reshape(N_TRI // 128, 128),
                                 sc_b.reshape(N_TRI // 128, 128))
    return (out_a.reshape(N_ENT, N_ENT), out_b.reshape(N_ENT, N_ENT))


# speedup vs baseline: 1.8297x; 1.0715x over previous
import functools
import math

import jax
import jax.numpy as jnp
from jax import lax
from jax.experimental import pallas as pl
from jax.experimental.pallas import tpu as pltpu
from jax.experimental.pallas import tpu_sc as plsc

N_ENT = 4096
N_REL = 512
N_TRI = 131072
DIM = 128
LANES = 16
NC = 2
NS = 16
NW = NC * NS
TRI_PER_W = N_TRI // NW
CHUNK = 128
N_CHUNK = TRI_PER_W // CHUNK
DENOM_INV = 1.0 / (3.0 * math.sqrt(DIM))

ROWS_PER_SC = N_ENT // NC
PASS_ROWS = 384
N_FULL = 5
TAIL_ROWS = ROWS_PER_SC - N_FULL * PASS_ROWS
ACC = PASS_ROWS * N_ENT
TRI_PER_T = N_TRI // NS
ZCHUNK = 4096

_mesh = plsc.VectorSubcoreMesh(core_axis_name="c", subcore_axis_name="s")


def _newton_sqrt(x):
    ib = lax.bitcast_convert_type(x, jnp.int32)
    ib = jnp.int32(0x5F3759DF) - lax.shift_right_arithmetic(ib, 1)
    y = lax.bitcast_convert_type(ib, jnp.float32)
    for _ in range(3):
        y = y * (1.5 - 0.5 * x * y * y)
    return x * y


def _score_body(ent_a, ent_b, rel_a, rel_b, h_a, h_b, t_a, t_b, r_a, r_b,
                scores_a, scores_b,
                hidx, tidx, ridx, hrows0, trows0, rrows0,
                hrows1, trows1, rrows1, scorebuf, sem0, sem1):
    cid = lax.axis_index("c")
    sid = lax.axis_index("s")
    wid = sid * NC + cid
    rowbase = wid * (TRI_PER_W // 128)

    lane = lax.broadcasted_iota(jnp.int32, (LANES,), 0)
    perms = [jnp.mod(lane + s, LANES).reshape(LANES, 1) for s in (8, 4, 2, 1)]
    dnums = lax.GatherDimensionNumbers(
        offset_dims=(), collapsed_slice_dims=(0,), start_index_map=(0,))

    def _permute(x, perm):
        return lax.gather(x, perm, dnums, slice_sizes=(1,),
                          mode=lax.GatherScatterMode.PROMISE_IN_BOUNDS)

    for ent_s, rel_s, h_s, t_s, r_s, scores_s in (
            (ent_a, rel_a, h_a, t_a, r_a, scores_a),
            (ent_b, rel_b, h_b, t_b, r_b, scores_b)):
        pltpu.sync_copy(h_s.at[pl.ds(rowbase, N_CHUNK)], hidx)
        pltpu.sync_copy(t_s.at[pl.ds(rowbase, N_CHUNK)], tidx)
        pltpu.sync_copy(r_s.at[pl.ds(rowbase, N_CHUNK)], ridx)

        def _fire(cidx, hrows, trows, rrows, sem):
            pltpu.async_copy(ent_s.at[hidx.at[cidx]], hrows, sem)
            pltpu.async_copy(ent_s.at[tidx.at[cidx]], trows, sem)
            pltpu.async_copy(rel_s.at[ridx.at[cidx]], rrows, sem)

        def _drain(cidx, hrows, trows, rrows, sem):
            pltpu.make_async_copy(ent_s.at[hidx.at[cidx]], hrows, sem).wait()
            pltpu.make_async_copy(ent_s.at[tidx.at[cidx]], trows, sem).wait()
            pltpu.make_async_copy(rel_s.at[ridx.at[cidx]], rrows, sem).wait()

        def _compute(cidx, hrows, trows, rrows):
            @plsc.parallel_loop(0, CHUNK // LANES)
            def _group(g):
                @plsc.parallel_loop(0, LANES,
                                    carry=jnp.zeros((LANES,), jnp.float32))
                def vec(u, vec_c):
                    i = g * LANES + u
                    acc0 = jnp.zeros((LANES,), jnp.float32)
                    acc1 = jnp.zeros((LANES,), jnp.float32)
                    bc = lambda v: lax.bitcast_convert_type(v, jnp.float32)
                    for j in range(DIM // 32):
                        wh = hrows[i, pl.ds(j * LANES, LANES)]
                        wr = rrows[i, pl.ds(j * LANES, LANES)]
                        wt = trows[i, pl.ds(j * LANES, LANES)]
                        da = (bc(wh) + bc(wr)) - bc(wt)
                        db = (bc(lax.shift_left(wh, 16))
                              + bc(lax.shift_left(wr, 16))) \
                            - bc(lax.shift_left(wt, 16))
                        acc0 = acc0 + da * da
                        acc1 = acc1 + db * db
                    acc = acc0 + acc1
                    for perm in perms:
                        acc = acc + _permute(acc, perm)
                    return jnp.where(lane == u, acc, vec_c)
                score = 1.0 - _newton_sqrt(vec) * DENOM_INV
                scorebuf[pl.ds(cidx * CHUNK + g * LANES, LANES)] = score

        _fire(0, hrows0, trows0, rrows0, sem0)

        @pl.loop(0, N_CHUNK // 2)
        def _chunk(k):
            c0 = 2 * k
            _fire(c0 + 1, hrows1, trows1, rrows1, sem1)
            _drain(c0, hrows0, trows0, rrows0, sem0)
            _compute(c0, hrows0, trows0, rrows0)

            @pl.when(k < N_CHUNK // 2 - 1)
            def _():
                _fire(c0 + 2, hrows0, trows0, rrows0, sem0)

            _drain(c0 + 1, hrows1, trows1, rrows1, sem1)
            _compute(c0 + 1, hrows1, trows1, rrows1)

        pltpu.sync_copy(scorebuf,
                        scores_s.at[pl.ds(wid * TRI_PER_W, TRI_PER_W)])


_score_call = pl.kernel(
    _score_body,
    out_type=(jax.ShapeDtypeStruct((N_TRI,), jnp.float32),
              jax.ShapeDtypeStruct((N_TRI,), jnp.float32)),
    mesh=_mesh,
    scratch_types=[
        pltpu.VMEM((N_CHUNK, CHUNK), jnp.int32),
        pltpu.VMEM((N_CHUNK, CHUNK), jnp.int32),
        pltpu.VMEM((N_CHUNK, CHUNK), jnp.int32),
        pltpu.VMEM((CHUNK, DIM), jnp.int32),
        pltpu.VMEM((CHUNK, DIM), jnp.int32),
        pltpu.VMEM((CHUNK, DIM), jnp.int32),
        pltpu.VMEM((CHUNK, DIM), jnp.int32),
        pltpu.VMEM((CHUNK, DIM), jnp.int32),
        pltpu.VMEM((CHUNK, DIM), jnp.int32),
        pltpu.VMEM((TRI_PER_W,), jnp.float32),
        pltpu.SemaphoreType.DMA,
        pltpu.SemaphoreType.DMA,
    ],
)


N_SCHUNK = TRI_PER_T // CHUNK


def _scatter_body(h_a, h_b, t_a, t_b, s_a, s_b, out_a, out_b,
                  acc, gidx, vals, idxbuf, zeros_v, sem, zsem):
    cid = lax.axis_index("c")
    sid = lax.axis_index("s")
    tb = sid * (TRI_PER_T // CHUNK)

    lane = lax.broadcasted_iota(jnp.int32, (LANES,), 0)
    dump = jnp.int32(ACC) + lane * 8

    @pl.loop(0, ZCHUNK // LANES)
    def _zinit(k):
        zeros_v[pl.ds(k * LANES, LANES)] = jnp.zeros((LANES,), jnp.float32)

    for h_s, t_s, s_s, out_s in ((h_a, t_a, s_a, out_a),
                                 (h_b, t_b, s_b, out_b)):
        cp0 = pltpu.async_copy(h_s.at[pl.ds(tb, N_SCHUNK)], idxbuf, sem)
        cp1 = pltpu.async_copy(t_s.at[pl.ds(tb, N_SCHUNK)], gidx, zsem)
        cp2 = pltpu.async_copy(s_s.at[pl.ds(tb, N_SCHUNK)], vals, sem)
        cp0.wait()
        cp1.wait()
        cp2.wait()

        @plsc.parallel_loop(0, N_SCHUNK)
        def _pre(j):
            for g in range(CHUNK // LANES):
                hv = idxbuf[j, pl.ds(g * LANES, LANES)]
                tv = gidx[j, pl.ds(g * LANES, LANES)]
                gidx[j, pl.ds(g * LANES, LANES)] = hv * N_ENT + tv

        def _pass_step(base, nrows):
            stripe = nrows * N_ENT // NS
            acc_n = nrows * N_ENT
            zcps = [pltpu.async_copy(
                        zeros_v,
                        acc.at[pl.ds(sid * stripe + z * ZCHUNK, ZCHUNK)],
                        zsem)
                    for z in range(stripe // ZCHUNK)]

            @plsc.parallel_loop(0, N_SCHUNK)
            def _idx(j):
                for g in range(CHUNK // LANES):
                    gv = gidx[j, pl.ds(g * LANES, LANES)] - base
                    m = (gv >= 0) & (gv < acc_n)
                    idxbuf[j, pl.ds(g * LANES, LANES)] = jnp.where(m, gv,
                                                                   dump)

            for cp in zcps:
                cp.wait()
            plsc.subcore_barrier()

            cps = [pltpu.async_copy(vals.at[j], acc.at[idxbuf.at[j]], sem,
                                    add=True)
                   for j in range(N_SCHUNK)]
            for cp in cps:
                cp.wait()

            plsc.subcore_barrier()
            pl.delay(1000)
            pltpu.sync_copy(acc.at[pl.ds(sid * stripe, stripe)],
                            out_s.at[pl.ds(base + sid * stripe, stripe)])

        @pl.loop(0, N_FULL)
        def _pass(p):
            _pass_step(cid * (ROWS_PER_SC * N_ENT) + p * (PASS_ROWS * N_ENT),
                       PASS_ROWS)

        _pass_step(cid * (ROWS_PER_SC * N_ENT) + N_FULL * (PASS_ROWS * N_ENT),
                   TAIL_ROWS)


_scatter_call = pl.kernel(
    _scatter_body,
    out_type=(jax.ShapeDtypeStruct((N_ENT * N_ENT,), jnp.float32),
              jax.ShapeDtypeStruct((N_ENT * N_ENT,), jnp.float32)),
    mesh=_mesh,
    scratch_types=[
        pltpu.VMEM_SHARED((ACC + 128,), jnp.float32),
        pltpu.VMEM((N_SCHUNK, CHUNK), jnp.int32),
        pltpu.VMEM((N_SCHUNK, CHUNK), jnp.float32),
        pltpu.VMEM((N_SCHUNK, CHUNK), jnp.int32),
        pltpu.VMEM((ZCHUNK,), jnp.float32),
        pltpu.SemaphoreType.DMA,
        pltpu.SemaphoreType.DMA,
    ],
)


def kernel(entity_emb_sr, entity_emb_tg, relation_emb_sr, relation_emb_tg,
           head_sr, tail_sr, relation_sr, head_tg, tail_tg, relation_tg):
    def _i2(a):
        return a.astype(jnp.int32).reshape(N_TRI // 128, 128)

    ha, hb = _i2(head_sr), _i2(head_tg)
    ta, tb = _i2(tail_sr), _i2(tail_tg)
    ra, rb = _i2(relation_sr), _i2(relation_tg)
    def _pack(x):
        n = x.shape[0]
        w = lax.bitcast_convert_type(
            x.astype(jnp.bfloat16).reshape(n, DIM // 2, 2), jnp.int32)
        return jnp.concatenate(
            [w, jnp.zeros((n, DIM // 2), jnp.int32)], axis=1)

    sc_a, sc_b = _score_call(_pack(entity_emb_sr), _pack(entity_emb_tg),
                             _pack(relation_emb_sr), _pack(relation_emb_tg),
                             ha, hb, ta, tb, ra, rb)
    out_a, out_b = _scatter_call(ha, hb, ta, tb,
                                 sc_a.reshape(N_TRI // 128, 128),
                                 sc_b.reshape(N_TRI // 128, 128))
    return (out_a.reshape(N_ENT, N_ENT), out_b.reshape(N_ENT, N_ENT))
